# Initial kernel scaffold; baseline (speedup 1.0000x reference)
import jax
import jax.numpy as jnp
from jax.experimental import pallas as pl


def kernel(obs, neis, self_labels, nei_labels, modes, W_obs, b_obs, W_nei, b_nei, W_mode, b_mode):
    raise NotImplementedError("write your pallas kernel here")


# speedup vs baseline: 1.5762x; 1.5762x over previous
import jax
import jax.numpy as jnp
from jax.experimental import pallas as pl
from jax.experimental.pallas import tpu as pltpu

NUM_CLASS = 8
EMBED = 256
NUM_MODES = 20
D_IN = 100

R_NEI = 512
R_OBS = 256


def _nei_body(neis_ref, lab_ref, W_ref, b_ref, out_ref):
    x = neis_ref[...]
    t = jnp.where(x >= 0, 1.0 / (x + 0.0001), 1.0 / (x - 0.0001))
    lab = lab_ref[...]
    acc = jnp.zeros((t.shape[0], EMBED), jnp.float32)
    for i in range(NUM_CLASS + 1):
        f = jax.lax.dot_general(t, W_ref[i], (((1,), (1,)), ((), ())),
                                preferred_element_type=jnp.float32)
        f = f + b_ref[i][None, :]
        acc = jnp.where(lab == i, f, acc)
    out_ref[...] = acc


def _obs_body(obs_ref, lab_ref, Wo_ref, bo_ref, Wm_ref, bm_ref, modes_ref,
              y1_ref, mp_ref):
    o = obs_ref[...]
    lab = lab_ref[...]
    x = jnp.zeros((o.shape[0], EMBED), jnp.float32)
    for i in range(NUM_CLASS):
        f = jax.lax.dot_general(o, Wo_ref[i], (((1,), (1,)), ((), ())),
                                preferred_element_type=jnp.float32)
        f = f + bo_ref[i][None, :]
        x = jnp.where(lab == i, f, x)
    W1 = Wm_ref[:, :EMBED]
    y1 = jax.lax.dot_general(x, W1, (((1,), (1,)), ((), ())),
                             preferred_element_type=jnp.float32)
    y1_ref[...] = y1 + bm_ref[...]
    W2 = Wm_ref[:, EMBED:]
    mp_ref[...] = jax.lax.dot_general(modes_ref[...], W2,
                                      (((1,), (1,)), ((), ())),
                                      preferred_element_type=jnp.float32)


def _assemble_body(y1_ref, lab_ref, mp_ref, out_ref):
    lab = lab_ref[...]
    y1 = y1_ref[...]
    cy = jnp.concatenate([y1] * NUM_MODES, axis=1)
    acc = cy
    for c in range(NUM_CLASS):
        acc = jnp.where(lab == c, cy + mp_ref[c][None, :], acc)
    out_ref[...] = acc


def kernel(obs, neis, self_labels, nei_labels, modes,
           W_obs, b_obs, W_nei, b_nei, W_mode, b_mode):
    B = obs.shape[0]
    N = neis.shape[1]
    BN = B * N
    neis_f = neis.reshape(BN, D_IN)
    nlab2 = nei_labels.reshape(BN, 1)
    obs_f = obs.reshape(B, D_IN)
    slab2 = self_labels.reshape(B, 1)
    modes_r = modes.reshape(NUM_CLASS * NUM_MODES, EMBED)
    bm2 = b_mode.reshape(1, EMBED)

    nei_feats = pl.pallas_call(
        _nei_body,
        grid=(BN // R_NEI,),
        in_specs=[
            pl.BlockSpec((R_NEI, D_IN), lambda g: (g, 0)),
            pl.BlockSpec((R_NEI, 1), lambda g: (g, 0)),
            pl.BlockSpec((NUM_CLASS + 1, EMBED, D_IN), lambda g: (0, 0, 0)),
            pl.BlockSpec((NUM_CLASS + 1, EMBED), lambda g: (0, 0)),
        ],
        out_specs=pl.BlockSpec((R_NEI, EMBED), lambda g: (g, 0)),
        out_shape=jax.ShapeDtypeStruct((BN, EMBED), jnp.float32),
    )(neis_f, nlab2, W_nei, b_nei)

    y1, mp = pl.pallas_call(
        _obs_body,
        grid=(B // R_OBS,),
        in_specs=[
            pl.BlockSpec((R_OBS, D_IN), lambda g: (g, 0)),
            pl.BlockSpec((R_OBS, 1), lambda g: (g, 0)),
            pl.BlockSpec((NUM_CLASS, EMBED, D_IN), lambda g: (0, 0, 0)),
            pl.BlockSpec((NUM_CLASS, EMBED), lambda g: (0, 0)),
            pl.BlockSpec((EMBED, 2 * EMBED), lambda g: (0, 0)),
            pl.BlockSpec((1, EMBED), lambda g: (0, 0)),
            pl.BlockSpec((NUM_CLASS * NUM_MODES, EMBED), lambda g: (0, 0)),
        ],
        out_specs=[
            pl.BlockSpec((R_OBS, EMBED), lambda g: (g, 0)),
            pl.BlockSpec((NUM_CLASS * NUM_MODES, EMBED), lambda g: (0, 0)),
        ],
        out_shape=[
            jax.ShapeDtypeStruct((B, EMBED), jnp.float32),
            jax.ShapeDtypeStruct((NUM_CLASS * NUM_MODES, EMBED), jnp.float32),
        ],
    )(obs_f, slab2, W_obs, b_obs, W_mode, bm2, modes_r)

    mp2 = mp.reshape(NUM_CLASS, NUM_MODES * EMBED)
    out1 = pl.pallas_call(
        _assemble_body,
        grid=(B // R_OBS,),
        in_specs=[
            pl.BlockSpec((R_OBS, EMBED), lambda g: (g, 0)),
            pl.BlockSpec((R_OBS, 1), lambda g: (g, 0)),
            pl.BlockSpec((NUM_CLASS, NUM_MODES * EMBED), lambda g: (0, 0)),
        ],
        out_specs=pl.BlockSpec((R_OBS, NUM_MODES * EMBED), lambda g: (g, 0)),
        out_shape=jax.ShapeDtypeStruct((B, NUM_MODES * EMBED), jnp.float32),
    )(y1, slab2, mp2)

    return (out1.reshape(B, NUM_MODES, EMBED), nei_feats.reshape(B, N, EMBED))


# speedup vs baseline: 1.5780x; 1.0011x over previous
import jax
import jax.numpy as jnp
from jax.experimental import pallas as pl
from jax.experimental.pallas import tpu as pltpu

NUM_CLASS = 8
EMBED = 256
NUM_MODES = 20
D_IN = 100

R_NEI = 512
R_OBS = 256


def _nei_body(neis_ref, lab_ref, W_ref, b_ref, out_ref):
    x = neis_ref[...]
    t = jnp.where(x >= 0, 1.0 / (x + 0.0001), 1.0 / (x - 0.0001))
    t = t.astype(jnp.bfloat16)
    lab = lab_ref[...]
    f_all = jax.lax.dot_general(t, W_ref[...], (((1,), (1,)), ((), ())),
                                preferred_element_type=jnp.float32)
    acc = jnp.zeros((t.shape[0], EMBED), jnp.float32)
    for i in range(NUM_CLASS + 1):
        f = f_all[:, i * EMBED:(i + 1) * EMBED] + b_ref[i][None, :]
        acc = jnp.where(lab == i, f, acc)
    out_ref[...] = acc


def _obs_body(obs_ref, lab_ref, Wo_ref, bo_ref, Wm_ref, bm_ref, modes_ref,
              y1_ref, mp_ref):
    o = obs_ref[...]
    lab = lab_ref[...]
    x = jnp.zeros((o.shape[0], EMBED), jnp.float32)
    for i in range(NUM_CLASS):
        f = jax.lax.dot_general(o, Wo_ref[i], (((1,), (1,)), ((), ())),
                                preferred_element_type=jnp.float32)
        f = f + bo_ref[i][None, :]
        x = jnp.where(lab == i, f, x)
    W1 = Wm_ref[:, :EMBED]
    y1 = jax.lax.dot_general(x, W1, (((1,), (1,)), ((), ())),
                             preferred_element_type=jnp.float32)
    y1_ref[...] = y1 + bm_ref[...]
    W2 = Wm_ref[:, EMBED:]
    mp_ref[...] = jax.lax.dot_general(modes_ref[...], W2,
                                      (((1,), (1,)), ((), ())),
                                      preferred_element_type=jnp.float32)


def _assemble_body(y1_ref, lab_ref, mp_ref, out_ref):
    lab = lab_ref[...]
    y1 = y1_ref[...]
    cy = jnp.concatenate([y1] * NUM_MODES, axis=1)
    acc = cy
    for c in range(NUM_CLASS):
        acc = jnp.where(lab == c, cy + mp_ref[c][None, :], acc)
    out_ref[...] = acc


def kernel(obs, neis, self_labels, nei_labels, modes,
           W_obs, b_obs, W_nei, b_nei, W_mode, b_mode):
    B = obs.shape[0]
    N = neis.shape[1]
    BN = B * N
    neis_f = neis.reshape(BN, D_IN)
    nlab2 = nei_labels.reshape(BN, 1)
    obs_f = obs.reshape(B, D_IN)
    slab2 = self_labels.reshape(B, 1)
    modes_r = modes.reshape(NUM_CLASS * NUM_MODES, EMBED)
    bm2 = b_mode.reshape(1, EMBED)

    W_nei_c = W_nei.reshape((NUM_CLASS + 1) * EMBED, D_IN).astype(jnp.bfloat16)
    nei_feats = pl.pallas_call(
        _nei_body,
        grid=(BN // R_NEI,),
        in_specs=[
            pl.BlockSpec((R_NEI, D_IN), lambda g: (g, 0)),
            pl.BlockSpec((R_NEI, 1), lambda g: (g, 0)),
            pl.BlockSpec(((NUM_CLASS + 1) * EMBED, D_IN), lambda g: (0, 0)),
            pl.BlockSpec((NUM_CLASS + 1, EMBED), lambda g: (0, 0)),
        ],
        out_specs=pl.BlockSpec((R_NEI, EMBED), lambda g: (g, 0)),
        out_shape=jax.ShapeDtypeStruct((BN, EMBED), jnp.float32),
    )(neis_f, nlab2, W_nei_c, b_nei)

    y1, mp = pl.pallas_call(
        _obs_body,
        grid=(B // R_OBS,),
        in_specs=[
            pl.BlockSpec((R_OBS, D_IN), lambda g: (g, 0)),
            pl.BlockSpec((R_OBS, 1), lambda g: (g, 0)),
            pl.BlockSpec((NUM_CLASS, EMBED, D_IN), lambda g: (0, 0, 0)),
            pl.BlockSpec((NUM_CLASS, EMBED), lambda g: (0, 0)),
            pl.BlockSpec((EMBED, 2 * EMBED), lambda g: (0, 0)),
            pl.BlockSpec((1, EMBED), lambda g: (0, 0)),
            pl.BlockSpec((NUM_CLASS * NUM_MODES, EMBED), lambda g: (0, 0)),
        ],
        out_specs=[
            pl.BlockSpec((R_OBS, EMBED), lambda g: (g, 0)),
            pl.BlockSpec((NUM_CLASS * NUM_MODES, EMBED), lambda g: (0, 0)),
        ],
        out_shape=[
            jax.ShapeDtypeStruct((B, EMBED), jnp.float32),
            jax.ShapeDtypeStruct((NUM_CLASS * NUM_MODES, EMBED), jnp.float32),
        ],
    )(obs_f, slab2, W_obs, b_obs, W_mode, bm2, modes_r)

    mp2 = mp.reshape(NUM_CLASS, NUM_MODES * EMBED)
    out1 = pl.pallas_call(
        _assemble_body,
        grid=(B // R_OBS,),
        in_specs=[
            pl.BlockSpec((R_OBS, EMBED), lambda g: (g, 0)),
            pl.BlockSpec((R_OBS, 1), lambda g: (g, 0)),
            pl.BlockSpec((NUM_CLASS, NUM_MODES * EMBED), lambda g: (0, 0)),
        ],
        out_specs=pl.BlockSpec((R_OBS, NUM_MODES * EMBED), lambda g: (g, 0)),
        out_shape=jax.ShapeDtypeStruct((B, NUM_MODES * EMBED), jnp.float32),
    )(y1, slab2, mp2)

    return (out1.reshape(B, NUM_MODES, EMBED), nei_feats.reshape(B, N, EMBED))


# speedup vs baseline: 2.0868x; 1.3225x over previous
import jax
import jax.numpy as jnp
from jax.experimental import pallas as pl
from jax.experimental.pallas import tpu as pltpu

NUM_CLASS = 8
NUM_EXP = 9
EMBED = 256
NUM_MODES = 20
D_IN = 100


def _nei_body(neis_ref, lab_ref, W_ref, bT_ref, out_ref):
    def body(n, carry):
        nm = neis_ref[n]
        t = jnp.where(nm >= 0, 1.0 / (nm + 0.0001), 1.0 / (nm - 0.0001))
        t = t.astype(jnp.bfloat16)
        f_all = jax.lax.dot_general(W_ref[...], t, (((1,), (0,)), ((), ())),
                                    preferred_element_type=jnp.float32)
        lab = lab_ref[n]
        acc = jnp.zeros((EMBED, t.shape[1]), jnp.float32)
        for e in range(NUM_EXP):
            f = f_all[e * EMBED:(e + 1) * EMBED, :] + bT_ref[:, e:e + 1]
            acc = jnp.where(lab == e, f, acc)
        out_ref[:, n, :] = jnp.transpose(acc, (1, 0))
        return carry

    jax.lax.fori_loop(0, 32, body, 0)


def _obs_body(obs_ref, lab_ref, Wo_ref, boT_ref, Wm_ref, bm_ref, y1_ref):
    om = obs_ref[...]
    xf = jax.lax.dot_general(Wo_ref[...], om, (((1,), (0,)), ((), ())),
                             preferred_element_type=jnp.float32)
    lab = lab_ref[...]
    x = jnp.zeros((EMBED, om.shape[1]), jnp.float32)
    for c in range(NUM_CLASS):
        f = xf[c * EMBED:(c + 1) * EMBED, :] + boT_ref[:, c:c + 1]
        x = jnp.where(lab == c, f, x)
    W1 = Wm_ref[:, :EMBED]
    y1 = jax.lax.dot_general(W1, x, (((1,), (0,)), ((), ())),
                             preferred_element_type=jnp.float32)
    y1_ref[...] = y1 + bm_ref[...]


def _asm_body(y1_ref, lab_ref, mp_ref, out_ref):
    lab = lab_ref[...]
    iot = jax.lax.broadcasted_iota(jnp.int32, (NUM_CLASS, lab.shape[1]), 0)
    oh = (iot == lab).astype(jnp.float32)
    mp_m = mp_ref[...]
    delta = jax.lax.dot_general(mp_m, oh, (((0,), (0,)), ((), ())),
                                preferred_element_type=jnp.float32)
    out_ref[0] = jnp.transpose(y1_ref[...] + delta, (1, 0))


def kernel(obs, neis, self_labels, nei_labels, modes,
           W_obs, b_obs, W_nei, b_nei, W_mode, b_mode):
    B = obs.shape[0]
    N = neis.shape[1]
    neis3 = jnp.transpose(neis, (1, 2, 3, 0)).reshape(N, D_IN, B)
    obs_m = jnp.transpose(obs, (1, 2, 0)).reshape(D_IN, B)
    nlabT = jnp.transpose(nei_labels, (1, 0)).reshape(N, 1, B)
    slab = self_labels.reshape(1, B)
    W9 = W_nei.reshape(NUM_EXP * EMBED, D_IN).astype(jnp.bfloat16)
    bT = jnp.transpose(b_nei)
    Wo = W_obs.reshape(NUM_CLASS * EMBED, D_IN)
    boT = jnp.transpose(b_obs)
    bm = b_mode.reshape(EMBED, 1)
    W2 = W_mode[:, EMBED:]
    mp2 = jnp.einsum('cmf,ef->mce', modes, W2).reshape(NUM_MODES * NUM_CLASS,
                                                       EMBED)

    BT = 256
    neiT = pl.pallas_call(
        _nei_body,
        grid=(B // BT,),
        in_specs=[
            pl.BlockSpec((N, D_IN, BT), lambda g: (0, 0, g)),
            pl.BlockSpec((N, 1, BT), lambda g: (0, 0, g)),
            pl.BlockSpec((NUM_EXP * EMBED, D_IN), lambda g: (0, 0)),
            pl.BlockSpec((EMBED, NUM_EXP), lambda g: (0, 0)),
        ],
        out_specs=pl.BlockSpec((BT, N, EMBED), lambda g: (g, 0, 0)),
        out_shape=jax.ShapeDtypeStruct((B, N, EMBED), jnp.float32),
    )(neis3, nlabT, W9, bT)

    y1 = pl.pallas_call(
        _obs_body,
        grid=(1,),
        in_specs=[
            pl.BlockSpec((D_IN, B), lambda g: (0, 0)),
            pl.BlockSpec((1, B), lambda g: (0, 0)),
            pl.BlockSpec((NUM_CLASS * EMBED, D_IN), lambda g: (0, 0)),
            pl.BlockSpec((EMBED, NUM_CLASS), lambda g: (0, 0)),
            pl.BlockSpec((EMBED, 2 * EMBED), lambda g: (0, 0)),
            pl.BlockSpec((EMBED, 1), lambda g: (0, 0)),
        ],
        out_specs=pl.BlockSpec((EMBED, B), lambda g: (0, 0)),
        out_shape=jax.ShapeDtypeStruct((EMBED, B), jnp.float32),
    )(obs_m, slab, Wo, boT, W_mode, bm)

    out1T = pl.pallas_call(
        _asm_body,
        grid=(NUM_MODES,),
        in_specs=[
            pl.BlockSpec((EMBED, B), lambda g: (0, 0)),
            pl.BlockSpec((1, B), lambda g: (0, 0)),
            pl.BlockSpec((NUM_CLASS, EMBED), lambda g: (g, 0)),
        ],
        out_specs=pl.BlockSpec((1, B, EMBED), lambda g: (g, 0, 0)),
        out_shape=jax.ShapeDtypeStruct((NUM_MODES, B, EMBED), jnp.float32),
    )(y1, slab, mp2)

    out1 = jnp.transpose(out1T, (1, 0, 2))
    return (out1, neiT)


# speedup vs baseline: 2.4089x; 1.1543x over previous
import jax
import jax.numpy as jnp
from jax.experimental import pallas as pl
from jax.experimental.pallas import tpu as pltpu

NUM_CLASS = 8
NUM_EXP = 9
EMBED = 256
NUM_MODES = 20
D_IN = 100


def _nei_body(neis_ref, lab_ref, W_ref, bT_ref, out_ref):
    def one(n):
        nm = neis_ref[n]
        t = jnp.where(nm >= 0, 1.0 / (nm + 0.0001), 1.0 / (nm - 0.0001))
        t = t.astype(jnp.bfloat16)
        f_all = jax.lax.dot_general(W_ref[...], t, (((1,), (0,)), ((), ())),
                                    preferred_element_type=jnp.float32)
        lab = lab_ref[n]
        iot = jax.lax.broadcasted_iota(jnp.int32, (NUM_EXP, lab.shape[1]), 0)
        oh = (iot == lab).astype(jnp.float32)
        bsel = jax.lax.dot_general(bT_ref[...], oh, (((1,), (0,)), ((), ())),
                                   preferred_element_type=jnp.float32)
        acc = jnp.zeros((EMBED, t.shape[1]), jnp.float32)
        for e in range(NUM_EXP):
            acc = jnp.where(lab == e, f_all[e * EMBED:(e + 1) * EMBED, :], acc)
        out_ref[:, n, :] = jnp.transpose(acc + bsel, (1, 0))

    def body(i, carry):
        one(2 * i)
        one(2 * i + 1)
        return carry

    jax.lax.fori_loop(0, 16, body, 0)


def _obs_body(obs_ref, lab_ref, Wo_ref, boT_ref, Wm_ref, bm_ref, y1_ref):
    om = obs_ref[...]
    xf = jax.lax.dot_general(Wo_ref[...], om, (((1,), (0,)), ((), ())),
                             preferred_element_type=jnp.float32)
    lab = lab_ref[...]
    x = jnp.zeros((EMBED, om.shape[1]), jnp.float32)
    for c in range(NUM_CLASS):
        f = xf[c * EMBED:(c + 1) * EMBED, :] + boT_ref[:, c:c + 1]
        x = jnp.where(lab == c, f, x)
    W1 = Wm_ref[:, :EMBED]
    y1 = jax.lax.dot_general(W1, x, (((1,), (0,)), ((), ())),
                             preferred_element_type=jnp.float32)
    y1_ref[...] = y1 + bm_ref[...]


def _asm_body(y1_ref, lab_ref, mp_ref, out_ref):
    lab = lab_ref[...]
    iot = jax.lax.broadcasted_iota(jnp.int32, (NUM_CLASS, lab.shape[1]), 0)
    oh = (iot == lab).astype(jnp.float32)
    mp_m = mp_ref[...]
    delta = jax.lax.dot_general(mp_m, oh, (((0,), (0,)), ((), ())),
                                preferred_element_type=jnp.float32)
    out_ref[0] = jnp.transpose(y1_ref[...] + delta, (1, 0))


def kernel(obs, neis, self_labels, nei_labels, modes,
           W_obs, b_obs, W_nei, b_nei, W_mode, b_mode):
    B = obs.shape[0]
    N = neis.shape[1]
    neis3 = jnp.transpose(neis, (1, 2, 3, 0)).reshape(N, D_IN, B)
    obs_m = jnp.transpose(obs, (1, 2, 0)).reshape(D_IN, B)
    nlabT = jnp.transpose(nei_labels, (1, 0)).reshape(N, 1, B)
    slab = self_labels.reshape(1, B)
    W9 = W_nei.reshape(NUM_EXP * EMBED, D_IN).astype(jnp.bfloat16)
    bT = jnp.transpose(b_nei)
    Wo = W_obs.reshape(NUM_CLASS * EMBED, D_IN)
    boT = jnp.transpose(b_obs)
    bm = b_mode.reshape(EMBED, 1)
    W2 = W_mode[:, EMBED:]
    mp2 = jnp.einsum('cmf,ef->mce', modes, W2).reshape(NUM_MODES * NUM_CLASS,
                                                       EMBED)

    BT = 256
    neiT = pl.pallas_call(
        _nei_body,
        grid=(B // BT,),
        in_specs=[
            pl.BlockSpec((N, D_IN, BT), lambda g: (0, 0, g)),
            pl.BlockSpec((N, 1, BT), lambda g: (0, 0, g)),
            pl.BlockSpec((NUM_EXP * EMBED, D_IN), lambda g: (0, 0)),
            pl.BlockSpec((EMBED, NUM_EXP), lambda g: (0, 0)),
        ],
        out_specs=pl.BlockSpec((BT, N, EMBED), lambda g: (g, 0, 0)),
        out_shape=jax.ShapeDtypeStruct((B, N, EMBED), jnp.float32),
    )(neis3, nlabT, W9, bT)

    y1 = pl.pallas_call(
        _obs_body,
        grid=(1,),
        in_specs=[
            pl.BlockSpec((D_IN, B), lambda g: (0, 0)),
            pl.BlockSpec((1, B), lambda g: (0, 0)),
            pl.BlockSpec((NUM_CLASS * EMBED, D_IN), lambda g: (0, 0)),
            pl.BlockSpec((EMBED, NUM_CLASS), lambda g: (0, 0)),
            pl.BlockSpec((EMBED, 2 * EMBED), lambda g: (0, 0)),
            pl.BlockSpec((EMBED, 1), lambda g: (0, 0)),
        ],
        out_specs=pl.BlockSpec((EMBED, B), lambda g: (0, 0)),
        out_shape=jax.ShapeDtypeStruct((EMBED, B), jnp.float32),
    )(obs_m, slab, Wo, boT, W_mode, bm)

    out1T = pl.pallas_call(
        _asm_body,
        grid=(NUM_MODES,),
        in_specs=[
            pl.BlockSpec((EMBED, B), lambda g: (0, 0)),
            pl.BlockSpec((1, B), lambda g: (0, 0)),
            pl.BlockSpec((NUM_CLASS, EMBED), lambda g: (g, 0)),
        ],
        out_specs=pl.BlockSpec((1, B, EMBED), lambda g: (g, 0, 0)),
        out_shape=jax.ShapeDtypeStruct((NUM_MODES, B, EMBED), jnp.float32),
    )(y1, slab, mp2)

    out1 = jnp.transpose(out1T, (1, 0, 2))
    return (out1, neiT)
